# Initial kernel scaffold; baseline (speedup 1.0000x reference)
import jax
import jax.numpy as jnp
from jax.experimental import pallas as pl


def kernel(x_nchw, b00_w, b00_g, b00_b, b01_w, b01_g, b01_b, b10_w, b10_g, b10_b, b11_w, b11_g, b11_b, b12_w, b12_g, b12_b, b2_w, b2_g, b2_b, b3_w, b3_g, b3_b):
    raise NotImplementedError("write your pallas kernel here")


# speedup vs baseline: 3.9174x; 3.9174x over previous
import functools

import jax
import jax.numpy as jnp
from jax import lax
from jax.experimental import pallas as pl
from jax.experimental.pallas import tpu as pltpu

_EPS = 1e-5


def _finalize_bn(s_ref, ss_ref, g_ref, b_ref, count):
    inv = 1.0 / count
    s = jnp.sum(s_ref[...], axis=0)
    ss = jnp.sum(ss_ref[...], axis=0)
    mean = s * inv
    var = jnp.maximum(ss * inv - mean * mean, 0.0)
    scale = g_ref[...] * lax.rsqrt(var + _EPS)
    offset = b_ref[...] - mean * scale
    return scale, offset


def _taps_2d(a, scratch, wcol, w, pad, hw):
    zero = jnp.zeros((), a.dtype)
    scratch[0:pad, :] = jnp.zeros((pad, a.shape[1]), a.dtype)
    scratch[pad + hw:pad + hw + pad, :] = jnp.zeros((pad, a.shape[1]), a.dtype)
    scratch[pad:pad + hw, :] = a
    taps = []
    for dh in (-1, 0, 1):
        for dw in (-1, 0, 1):
            base = pad + dh * w + dw
            t = scratch[base:base + hw, :]
            if dw == -1:
                t = jnp.where(wcol >= 1, t, zero)
            elif dw == 1:
                t = jnp.where(wcol <= w - 2, t, zero)
            taps.append(t)
    return taps


def _k1(x_ref, wm_ref, wp_ref, hm_ref, hp_ref,
        s1_ref, ss1_ref, s3_ref, ss3_ref, psc, *, w, hw, pad):
    xb = x_ref[0]
    hm = jnp.dot(xb, wm_ref[...], preferred_element_type=jnp.float32)
    hp = jnp.dot(xb, wp_ref[...], preferred_element_type=jnp.float32)
    wcol = lax.broadcasted_iota(jnp.int32, (hw, 1), 0) % w
    taps = _taps_2d(hp, psc, wcol, w, pad, hw)
    pooled = functools.reduce(lambda a, b: a + b, taps) * (1.0 / 9.0)
    hm_ref[0] = hm.astype(jnp.bfloat16)
    hp_ref[0] = pooled.astype(jnp.bfloat16)
    s1_ref[0] = jnp.sum(hm, axis=0, keepdims=True)
    ss1_ref[0] = jnp.sum(hm * hm, axis=0, keepdims=True)
    s3_ref[0] = jnp.sum(pooled, axis=0, keepdims=True)
    ss3_ref[0] = jnp.sum(pooled * pooled, axis=0, keepdims=True)


def _conv3x3(a, w_ref, scratch, wcol, w, pad, hw):
    taps = _taps_2d(a, scratch, wcol, w, pad, hw)
    im = jnp.concatenate(taps, axis=1)
    return jnp.dot(im, w_ref[...], preferred_element_type=jnp.float32)


def _k2(hm_ref, hp_ref, s1_ref, ss1_ref, s3_ref, ss3_ref,
        g1_ref, b1_ref, g3_ref, b3_ref, w01_ref, w11_ref,
        c0_ref, c1_ref, o2_ref, o3_ref,
        sc0_ref, ssc0_ref, sc1_ref, ssc1_ref,
        sca, scb, *, w, hw, pad, cp, c, count):
    scale, offset = _finalize_bn(s1_ref, ss1_ref, g1_ref, b1_ref, count)
    act = jnp.maximum(hm_ref[0].astype(jnp.float32) * scale + offset, 0.0)
    actb = act.astype(jnp.bfloat16)
    o2_ref[0] = actb[:, 2 * cp:2 * cp + c]
    sc3, of3 = _finalize_bn(s3_ref, ss3_ref, g3_ref, b3_ref, count)
    o3_ref[0] = jnp.maximum(
        hp_ref[0].astype(jnp.float32) * sc3 + of3, 0.0).astype(jnp.bfloat16)
    wcol = lax.broadcasted_iota(jnp.int32, (hw, 1), 0) % w
    c0 = _conv3x3(actb[:, 0:cp], w01_ref, sca, wcol, w, pad, hw)
    c0_ref[0] = c0.astype(jnp.bfloat16)
    sc0_ref[0] = jnp.sum(c0, axis=0, keepdims=True)
    ssc0_ref[0] = jnp.sum(c0 * c0, axis=0, keepdims=True)
    c1 = _conv3x3(actb[:, cp:2 * cp], w11_ref, scb, wcol, w, pad, hw)
    c1_ref[0] = c1.astype(jnp.bfloat16)
    sc1_ref[0] = jnp.sum(c1, axis=0, keepdims=True)
    ssc1_ref[0] = jnp.sum(c1 * c1, axis=0, keepdims=True)


def _k3(c0_ref, c1_ref, sc0_ref, ssc0_ref, sc1_ref, ssc1_ref,
        g01_ref, b01_ref, g11_ref, b11_ref, w12_ref,
        o0_ref, c2_ref, sc2_ref, ssc2_ref, scr, *, w, hw, pad, count):
    s0, o0 = _finalize_bn(sc0_ref, ssc0_ref, g01_ref, b01_ref, count)
    o0_ref[0] = jnp.maximum(
        c0_ref[0].astype(jnp.float32) * s0 + o0, 0.0).astype(jnp.bfloat16)
    s1, of1 = _finalize_bn(sc1_ref, ssc1_ref, g11_ref, b11_ref, count)
    act = jnp.maximum(
        c1_ref[0].astype(jnp.float32) * s1 + of1, 0.0).astype(jnp.bfloat16)
    wcol = lax.broadcasted_iota(jnp.int32, (hw, 1), 0) % w
    c2 = _conv3x3(act, w12_ref, scr, wcol, w, hw=hw, pad=pad)
    c2_ref[0] = c2.astype(jnp.bfloat16)
    sc2_ref[0] = jnp.sum(c2, axis=0, keepdims=True)
    ssc2_ref[0] = jnp.sum(c2 * c2, axis=0, keepdims=True)


def _k4(c2_ref, sc2_ref, ssc2_ref, g12_ref, b12_ref, o1_ref, *, count):
    s2, of2 = _finalize_bn(sc2_ref, ssc2_ref, g12_ref, b12_ref, count)
    o1_ref[0] = jnp.maximum(
        c2_ref[0].astype(jnp.float32) * s2 + of2, 0.0).astype(jnp.bfloat16)


def _pad_vec(v, cp):
    return jnp.zeros((1, cp), jnp.float32).at[0, :v.shape[0]].set(v)


def kernel(x_nchw, b00_w, b00_g, b00_b, b01_w, b01_g, b01_b,
           b10_w, b10_g, b10_b, b11_w, b11_g, b11_b, b12_w, b12_g, b12_b,
           b2_w, b2_g, b2_b, b3_w, b3_g, b3_b):
    n, cin, h, w = x_nchw.shape
    c = b00_w.shape[-1]
    cp = 128
    hw = h * w
    pad = w + 1
    count = float(n * hw)
    bf = jnp.bfloat16
    f32 = jnp.float32

    xt = jnp.transpose(x_nchw.reshape(n, cin, hw), (0, 2, 1)).astype(bf)

    wm = (jnp.zeros((cin, 3 * cp), f32)
          .at[:, 0:c].set(b00_w.reshape(cin, c))
          .at[:, cp:cp + c].set(b10_w.reshape(cin, c))
          .at[:, 2 * cp:2 * cp + c].set(b2_w.reshape(cin, c))).astype(bf)
    wp = b3_w.reshape(cin, c).astype(bf)
    g1 = (jnp.zeros((1, 3 * cp), f32).at[0, 0:c].set(b00_g)
          .at[0, cp:cp + c].set(b10_g).at[0, 2 * cp:2 * cp + c].set(b2_g))
    b1 = (jnp.zeros((1, 3 * cp), f32).at[0, 0:c].set(b00_b)
          .at[0, cp:cp + c].set(b10_b).at[0, 2 * cp:2 * cp + c].set(b2_b))
    w01 = (jnp.zeros((3, 3, cp, c), f32).at[:, :, :c, :].set(b01_w)
           .reshape(9 * cp, c)).astype(bf)
    w11 = (jnp.zeros((3, 3, cp, cp), f32).at[:, :, :c, :c].set(b11_w)
           .reshape(9 * cp, cp)).astype(bf)
    w12 = (jnp.zeros((3, 3, cp, c), f32).at[:, :, :c, :].set(b12_w)
           .reshape(9 * cp, c)).astype(bf)

    par = pltpu.CompilerParams(dimension_semantics=("parallel",))

    hm, hp, s1, ss1, s3, ss3 = pl.pallas_call(
        functools.partial(_k1, w=w, hw=hw, pad=pad),
        grid=(n,),
        in_specs=[pl.BlockSpec((1, hw, cin), lambda i: (i, 0, 0)),
                  pl.BlockSpec((cin, 3 * cp), lambda i: (0, 0)),
                  pl.BlockSpec((cin, c), lambda i: (0, 0))],
        out_specs=(pl.BlockSpec((1, hw, 3 * cp), lambda i: (i, 0, 0)),
                   pl.BlockSpec((1, hw, c), lambda i: (i, 0, 0)),
                   pl.BlockSpec((1, 1, 3 * cp), lambda i: (i, 0, 0)),
                   pl.BlockSpec((1, 1, 3 * cp), lambda i: (i, 0, 0)),
                   pl.BlockSpec((1, 1, c), lambda i: (i, 0, 0)),
                   pl.BlockSpec((1, 1, c), lambda i: (i, 0, 0))),
        out_shape=(jax.ShapeDtypeStruct((n, hw, 3 * cp), bf),
                   jax.ShapeDtypeStruct((n, hw, c), bf),
                   jax.ShapeDtypeStruct((n, 1, 3 * cp), f32),
                   jax.ShapeDtypeStruct((n, 1, 3 * cp), f32),
                   jax.ShapeDtypeStruct((n, 1, c), f32),
                   jax.ShapeDtypeStruct((n, 1, c), f32)),
        scratch_shapes=[pltpu.VMEM((hw + 2 * pad, c), f32)],
        compiler_params=par,
    )(xt, wm, wp)

    stat = lambda cc: pl.BlockSpec((n, 1, cc), lambda i: (0, 0, 0))
    vec = lambda cc: pl.BlockSpec((1, cc), lambda i: (0, 0))

    c0, c1, o2, o3, sc0, ssc0, sc1, ssc1 = pl.pallas_call(
        functools.partial(_k2, w=w, hw=hw, pad=pad, cp=cp, c=c, count=count),
        grid=(n,),
        in_specs=[pl.BlockSpec((1, hw, 3 * cp), lambda i: (i, 0, 0)),
                  pl.BlockSpec((1, hw, c), lambda i: (i, 0, 0)),
                  stat(3 * cp), stat(3 * cp), stat(c), stat(c),
                  vec(3 * cp), vec(3 * cp), vec(c), vec(c),
                  pl.BlockSpec((9 * cp, c), lambda i: (0, 0)),
                  pl.BlockSpec((9 * cp, cp), lambda i: (0, 0))],
        out_specs=(pl.BlockSpec((1, hw, c), lambda i: (i, 0, 0)),
                   pl.BlockSpec((1, hw, cp), lambda i: (i, 0, 0)),
                   pl.BlockSpec((1, hw, c), lambda i: (i, 0, 0)),
                   pl.BlockSpec((1, hw, c), lambda i: (i, 0, 0)),
                   pl.BlockSpec((1, 1, c), lambda i: (i, 0, 0)),
                   pl.BlockSpec((1, 1, c), lambda i: (i, 0, 0)),
                   pl.BlockSpec((1, 1, cp), lambda i: (i, 0, 0)),
                   pl.BlockSpec((1, 1, cp), lambda i: (i, 0, 0))),
        out_shape=(jax.ShapeDtypeStruct((n, hw, c), bf),
                   jax.ShapeDtypeStruct((n, hw, cp), bf),
                   jax.ShapeDtypeStruct((n, hw, c), bf),
                   jax.ShapeDtypeStruct((n, hw, c), bf),
                   jax.ShapeDtypeStruct((n, 1, c), f32),
                   jax.ShapeDtypeStruct((n, 1, c), f32),
                   jax.ShapeDtypeStruct((n, 1, cp), f32),
                   jax.ShapeDtypeStruct((n, 1, cp), f32)),
        scratch_shapes=[pltpu.VMEM((hw + 2 * pad, cp), bf),
                        pltpu.VMEM((hw + 2 * pad, cp), bf)],
        compiler_params=par,
    )(hm, hp, s1, ss1, s3, ss3, g1, b1,
      b3_g.reshape(1, c), b3_b.reshape(1, c), w01, w11)

    g11 = _pad_vec(b11_g, cp)
    b11 = _pad_vec(b11_b, cp)
    o0, c2, sc2, ssc2 = pl.pallas_call(
        functools.partial(_k3, w=w, hw=hw, pad=pad, count=count),
        grid=(n,),
        in_specs=[pl.BlockSpec((1, hw, c), lambda i: (i, 0, 0)),
                  pl.BlockSpec((1, hw, cp), lambda i: (i, 0, 0)),
                  stat(c), stat(c), stat(cp), stat(cp),
                  vec(c), vec(c), vec(cp), vec(cp),
                  pl.BlockSpec((9 * cp, c), lambda i: (0, 0))],
        out_specs=(pl.BlockSpec((1, hw, c), lambda i: (i, 0, 0)),
                   pl.BlockSpec((1, hw, c), lambda i: (i, 0, 0)),
                   pl.BlockSpec((1, 1, c), lambda i: (i, 0, 0)),
                   pl.BlockSpec((1, 1, c), lambda i: (i, 0, 0))),
        out_shape=(jax.ShapeDtypeStruct((n, hw, c), bf),
                   jax.ShapeDtypeStruct((n, hw, c), bf),
                   jax.ShapeDtypeStruct((n, 1, c), f32),
                   jax.ShapeDtypeStruct((n, 1, c), f32)),
        scratch_shapes=[pltpu.VMEM((hw + 2 * pad, cp), bf)],
        compiler_params=par,
    )(c0, c1, sc0, ssc0, sc1, ssc1,
      b01_g.reshape(1, c), b01_b.reshape(1, c), g11, b11, w12)

    o1 = pl.pallas_call(
        functools.partial(_k4, count=count),
        grid=(n,),
        in_specs=[pl.BlockSpec((1, hw, c), lambda i: (i, 0, 0)),
                  stat(c), stat(c), vec(c), vec(c)],
        out_specs=pl.BlockSpec((1, hw, c), lambda i: (i, 0, 0)),
        out_shape=jax.ShapeDtypeStruct((n, hw, c), bf),
        compiler_params=par,
    )(c2, sc2, ssc2, b12_g.reshape(1, c), b12_b.reshape(1, c))

    y = jnp.concatenate([o0, o1, o2, o3], axis=-1)
    return jnp.transpose(y, (0, 2, 1)).astype(f32).reshape(n, 4 * c, h, w)


# speedup vs baseline: 4.0220x; 1.0267x over previous
import functools

import jax
import jax.numpy as jnp
from jax import lax
from jax.experimental import pallas as pl
from jax.experimental.pallas import tpu as pltpu

_EPS = 1e-5


def _finalize_bn(s_ref, ss_ref, g_ref, b_ref, count):
    inv = 1.0 / count
    s = jnp.sum(s_ref[...], axis=0)
    ss = jnp.sum(ss_ref[...], axis=0)
    mean = s * inv
    var = jnp.maximum(ss * inv - mean * mean, 0.0)
    scale = g_ref[...] * lax.rsqrt(var + _EPS)
    offset = b_ref[...] - mean * scale
    return scale, offset


def _taps_2d(a, scratch, ml, mr, w, pad, hw):
    scratch[0:pad, :] = jnp.zeros((pad, a.shape[1]), a.dtype)
    scratch[pad + hw:pad + hw + pad, :] = jnp.zeros((pad, a.shape[1]), a.dtype)
    scratch[pad:pad + hw, :] = a
    taps = []
    for dh in (-1, 0, 1):
        for dw in (-1, 0, 1):
            base = pad + dh * w + dw
            t = scratch[base:base + hw, :]
            if dw == -1:
                t = t * ml
            elif dw == 1:
                t = t * mr
            taps.append(t)
    return taps


def _k1(x_ref, wall_ref, mlf_ref, mrf_ref, hp_ref,
        s1_ref, ss1_ref, s3_ref, ss3_ref, psc, *, w, hw, pad, cp, c):
    xb = x_ref[0].astype(jnp.bfloat16)
    h = lax.dot_general(xb, wall_ref[...], (((0,), (0,)), ((), ())),
                        preferred_element_type=jnp.float32)
    hm = h[:, 0:3 * cp]
    hp = h[:, 3 * cp:3 * cp + c]
    taps = _taps_2d(hp, psc, mlf_ref[...], mrf_ref[...], w, pad, hw)
    pooled = functools.reduce(lambda x, y: x + y, taps) * (1.0 / 9.0)
    hp_ref[0] = pooled.astype(jnp.bfloat16)
    s1_ref[0] = jnp.sum(hm, axis=0, keepdims=True)
    ss1_ref[0] = jnp.sum(hm * hm, axis=0, keepdims=True)
    s3_ref[0] = jnp.sum(pooled, axis=0, keepdims=True)
    ss3_ref[0] = jnp.sum(pooled * pooled, axis=0, keepdims=True)


def _conv3x3(a, w_ref, scratch, ml, mr, w, pad, hw):
    taps = _taps_2d(a, scratch, ml, mr, w, pad, hw)
    im = jnp.concatenate(taps, axis=1)
    return jnp.dot(im, w_ref[...], preferred_element_type=jnp.float32)


def _k2(x_ref, wm_ref, hp_ref, s1_ref, ss1_ref, s3_ref, ss3_ref,
        g1_ref, b1_ref, g3_ref, b3_ref, w01_ref, w11_ref, mlb_ref, mrb_ref,
        c0_ref, c1_ref, o2_ref, o3_ref,
        sc0_ref, ssc0_ref, sc1_ref, ssc1_ref,
        sca, scb, *, w, hw, pad, cp, c, count):
    scale, offset = _finalize_bn(s1_ref, ss1_ref, g1_ref, b1_ref, count)
    xb = x_ref[0].astype(jnp.bfloat16)
    hm = lax.dot_general(xb, wm_ref[...], (((0,), (0,)), ((), ())),
                         preferred_element_type=jnp.float32)
    act = jnp.maximum(hm * scale + offset, 0.0)
    actb = act.astype(jnp.bfloat16)
    o2_ref[0] = actb[:, 2 * cp:2 * cp + c]
    sc3, of3 = _finalize_bn(s3_ref, ss3_ref, g3_ref, b3_ref, count)
    o3_ref[0] = jnp.maximum(
        hp_ref[0].astype(jnp.float32) * sc3 + of3, 0.0).astype(jnp.bfloat16)
    ml = mlb_ref[...]
    mr = mrb_ref[...]
    c0 = _conv3x3(actb[:, 0:cp], w01_ref, sca, ml, mr, w, pad, hw)
    c0_ref[0] = c0.astype(jnp.bfloat16)
    sc0_ref[0] = jnp.sum(c0, axis=0, keepdims=True)
    ssc0_ref[0] = jnp.sum(c0 * c0, axis=0, keepdims=True)
    c1 = _conv3x3(actb[:, cp:2 * cp], w11_ref, scb, ml, mr, w, pad, hw)
    c1_ref[0] = c1.astype(jnp.bfloat16)
    sc1_ref[0] = jnp.sum(c1, axis=0, keepdims=True)
    ssc1_ref[0] = jnp.sum(c1 * c1, axis=0, keepdims=True)


def _k3(c0_ref, c1_ref, sc0_ref, ssc0_ref, sc1_ref, ssc1_ref,
        g01_ref, b01_ref, g11_ref, b11_ref, w12_ref, mlb_ref, mrb_ref,
        o0_ref, c2_ref, sc2_ref, ssc2_ref, scr, *, w, hw, pad, cp, count):
    s0, o0 = _finalize_bn(sc0_ref, ssc0_ref, g01_ref, b01_ref, count)
    o0_ref[0] = jnp.maximum(
        c0_ref[0].astype(jnp.float32) * s0 + o0, 0.0).astype(jnp.bfloat16)
    s1, of1 = _finalize_bn(sc1_ref, ssc1_ref, g11_ref, b11_ref, count)
    act = jnp.maximum(
        c1_ref[0].astype(jnp.float32) * s1 + of1, 0.0).astype(jnp.bfloat16)
    c2 = _conv3x3(act, w12_ref, scr, mlb_ref[...], mrb_ref[...], w, pad, hw)
    c2_ref[0] = c2.astype(jnp.bfloat16)
    sc2_ref[0] = jnp.sum(c2, axis=0, keepdims=True)
    ssc2_ref[0] = jnp.sum(c2 * c2, axis=0, keepdims=True)


def _k4(o0_ref, c2_ref, o2_ref, o3_ref, sc2_ref, ssc2_ref,
        g12_ref, b12_ref, out_ref, *, c, count):
    s2, of2 = _finalize_bn(sc2_ref, ssc2_ref, g12_ref, b12_ref, count)
    a1 = jnp.maximum(c2_ref[0].astype(jnp.float32) * s2 + of2, 0.0)
    out_ref[0, 0:c, :] = jnp.transpose(o0_ref[0].astype(jnp.float32), (1, 0))
    out_ref[0, c:2 * c, :] = jnp.transpose(a1, (1, 0))
    out_ref[0, 2 * c:3 * c, :] = jnp.transpose(o2_ref[0].astype(jnp.float32),
                                               (1, 0))
    out_ref[0, 3 * c:4 * c, :] = jnp.transpose(o3_ref[0].astype(jnp.float32),
                                               (1, 0))


def _pad_vec(v, cc):
    return jnp.zeros((1, cc), jnp.float32).at[0, :v.shape[0]].set(v)


def kernel(x_nchw, b00_w, b00_g, b00_b, b01_w, b01_g, b01_b,
           b10_w, b10_g, b10_b, b11_w, b11_g, b11_b, b12_w, b12_g, b12_b,
           b2_w, b2_g, b2_b, b3_w, b3_g, b3_b):
    n, cin, h, w = x_nchw.shape
    c = b00_w.shape[-1]
    cp = 128
    hw = h * w
    pad = w + 1
    count = float(n * hw)
    bf = jnp.bfloat16
    f32 = jnp.float32

    x3 = x_nchw.reshape(n, cin, hw)

    wcol = (jnp.arange(hw, dtype=jnp.int32) % w).reshape(hw, 1)
    mlf = (wcol >= 1).astype(f32)
    mrf = (wcol <= w - 2).astype(f32)
    mlb = mlf.astype(bf)
    mrb = mrf.astype(bf)

    wall = (jnp.zeros((cin, 4 * cp), f32)
            .at[:, 0:c].set(b00_w.reshape(cin, c))
            .at[:, cp:cp + c].set(b10_w.reshape(cin, c))
            .at[:, 2 * cp:2 * cp + c].set(b2_w.reshape(cin, c))
            .at[:, 3 * cp:3 * cp + c].set(b3_w.reshape(cin, c))).astype(bf)
    wm = wall[:, 0:3 * cp]
    g1 = (jnp.zeros((1, 3 * cp), f32).at[0, 0:c].set(b00_g)
          .at[0, cp:cp + c].set(b10_g).at[0, 2 * cp:2 * cp + c].set(b2_g))
    b1 = (jnp.zeros((1, 3 * cp), f32).at[0, 0:c].set(b00_b)
          .at[0, cp:cp + c].set(b10_b).at[0, 2 * cp:2 * cp + c].set(b2_b))
    w01 = (jnp.zeros((3, 3, cp, c), f32).at[:, :, :c, :].set(b01_w)
           .reshape(9 * cp, c)).astype(bf)
    w11 = (jnp.zeros((3, 3, cp, cp), f32).at[:, :, :c, :c].set(b11_w)
           .reshape(9 * cp, cp)).astype(bf)
    w12 = (jnp.zeros((3, 3, cp, c), f32).at[:, :, :c, :].set(b12_w)
           .reshape(9 * cp, c)).astype(bf)

    par = pltpu.CompilerParams(dimension_semantics=("parallel",))
    img = lambda cc: pl.BlockSpec((1, hw, cc), lambda i: (i, 0, 0))
    istat = lambda cc: pl.BlockSpec((1, 1, cc), lambda i: (i, 0, 0))
    stat = lambda cc: pl.BlockSpec((n, 1, cc), lambda i: (0, 0, 0))
    vec = lambda cc: pl.BlockSpec((1, cc), lambda i: (0, 0))
    mask = pl.BlockSpec((hw, 1), lambda i: (0, 0))
    xspec = pl.BlockSpec((1, cin, hw), lambda i: (i, 0, 0))

    hp, s1, ss1, s3, ss3 = pl.pallas_call(
        functools.partial(_k1, w=w, hw=hw, pad=pad, cp=cp, c=c),
        grid=(n,),
        in_specs=[xspec, pl.BlockSpec((cin, 4 * cp), lambda i: (0, 0)),
                  mask, mask],
        out_specs=(img(c), istat(3 * cp), istat(3 * cp), istat(c), istat(c)),
        out_shape=(jax.ShapeDtypeStruct((n, hw, c), bf),
                   jax.ShapeDtypeStruct((n, 1, 3 * cp), f32),
                   jax.ShapeDtypeStruct((n, 1, 3 * cp), f32),
                   jax.ShapeDtypeStruct((n, 1, c), f32),
                   jax.ShapeDtypeStruct((n, 1, c), f32)),
        scratch_shapes=[pltpu.VMEM((hw + 2 * pad, c), f32)],
        compiler_params=par,
    )(x3, wall, mlf, mrf)

    c0, c1, o2, o3, sc0, ssc0, sc1, ssc1 = pl.pallas_call(
        functools.partial(_k2, w=w, hw=hw, pad=pad, cp=cp, c=c, count=count),
        grid=(n,),
        in_specs=[xspec, pl.BlockSpec((cin, 3 * cp), lambda i: (0, 0)),
                  img(c), stat(3 * cp), stat(3 * cp), stat(c), stat(c),
                  vec(3 * cp), vec(3 * cp), vec(c), vec(c),
                  pl.BlockSpec((9 * cp, c), lambda i: (0, 0)),
                  pl.BlockSpec((9 * cp, cp), lambda i: (0, 0)),
                  mask, mask],
        out_specs=(img(c), img(cp), img(c), img(c),
                   istat(c), istat(c), istat(cp), istat(cp)),
        out_shape=(jax.ShapeDtypeStruct((n, hw, c), bf),
                   jax.ShapeDtypeStruct((n, hw, cp), bf),
                   jax.ShapeDtypeStruct((n, hw, c), bf),
                   jax.ShapeDtypeStruct((n, hw, c), bf),
                   jax.ShapeDtypeStruct((n, 1, c), f32),
                   jax.ShapeDtypeStruct((n, 1, c), f32),
                   jax.ShapeDtypeStruct((n, 1, cp), f32),
                   jax.ShapeDtypeStruct((n, 1, cp), f32)),
        scratch_shapes=[pltpu.VMEM((hw + 2 * pad, cp), bf),
                        pltpu.VMEM((hw + 2 * pad, cp), bf)],
        compiler_params=par,
    )(x3, wm, hp, s1, ss1, s3, ss3, g1, b1,
      b3_g.reshape(1, c), b3_b.reshape(1, c), w01, w11, mlb, mrb)

    g11 = _pad_vec(b11_g, cp)
    b11 = _pad_vec(b11_b, cp)
    o0, c2, sc2, ssc2 = pl.pallas_call(
        functools.partial(_k3, w=w, hw=hw, pad=pad, cp=cp, count=count),
        grid=(n,),
        in_specs=[img(c), img(cp),
                  stat(c), stat(c), stat(cp), stat(cp),
                  vec(c), vec(c), vec(cp), vec(cp),
                  pl.BlockSpec((9 * cp, c), lambda i: (0, 0)),
                  mask, mask],
        out_specs=(img(c), img(c), istat(c), istat(c)),
        out_shape=(jax.ShapeDtypeStruct((n, hw, c), bf),
                   jax.ShapeDtypeStruct((n, hw, c), bf),
                   jax.ShapeDtypeStruct((n, 1, c), f32),
                   jax.ShapeDtypeStruct((n, 1, c), f32)),
        scratch_shapes=[pltpu.VMEM((hw + 2 * pad, cp), bf)],
        compiler_params=par,
    )(c0, c1, sc0, ssc0, sc1, ssc1,
      b01_g.reshape(1, c), b01_b.reshape(1, c), g11, b11, w12, mlb, mrb)

    out = pl.pallas_call(
        functools.partial(_k4, c=c, count=count),
        grid=(n,),
        in_specs=[img(c), img(c), img(c), img(c),
                  stat(c), stat(c), vec(c), vec(c)],
        out_specs=pl.BlockSpec((1, 4 * c, hw), lambda i: (i, 0, 0)),
        out_shape=jax.ShapeDtypeStruct((n, 4 * c, hw), f32),
        compiler_params=par,
    )(o0, c2, o2, o3, sc2, ssc2, b12_g.reshape(1, c), b12_b.reshape(1, c))

    return out.reshape(n, 4 * c, h, w)


# speedup vs baseline: 4.5450x; 1.1300x over previous
import functools

import jax
import jax.numpy as jnp
from jax import lax
from jax.experimental import pallas as pl
from jax.experimental.pallas import tpu as pltpu

_EPS = 1e-5
_IPP = 4


def _finalize_bn(s_ref, ss_ref, g_ref, b_ref, count):
    inv = 1.0 / count
    s = jnp.sum(s_ref[...], axis=0)
    ss = jnp.sum(ss_ref[...], axis=0)
    mean = s * inv
    var = jnp.maximum(ss * inv - mean * mean, 0.0)
    scale = g_ref[...] * lax.rsqrt(var + _EPS)
    offset = b_ref[...] - mean * scale
    return scale, offset


def _taps_2d(a, scratch, ml, mr, w, pad, hw):
    scratch[0:pad, :] = jnp.zeros((pad, a.shape[1]), a.dtype)
    scratch[pad + hw:pad + hw + pad, :] = jnp.zeros((pad, a.shape[1]), a.dtype)
    scratch[pad:pad + hw, :] = a
    taps = []
    for dh in (-1, 0, 1):
        for dw in (-1, 0, 1):
            base = pad + dh * w + dw
            t = scratch[base:base + hw, :]
            if dw == -1:
                t = t * ml
            elif dw == 1:
                t = t * mr
            taps.append(t)
    return taps


def _acc(tot, v):
    return v if tot is None else tot + v


def _k1(x_ref, wall_ref, mlf_ref, mrf_ref, hp_ref,
        s1_ref, ss1_ref, s3_ref, ss3_ref, psc, *, w, hw, pad, cp, c):
    ml = mlf_ref[...]
    mr = mrf_ref[...]
    s1 = ss1 = s3 = ss3 = None
    for j in range(_IPP):
        xb = x_ref[j].astype(jnp.bfloat16)
        h = lax.dot_general(xb, wall_ref[...], (((0,), (0,)), ((), ())),
                            preferred_element_type=jnp.float32)
        hm = h[:, 0:3 * cp]
        hp = h[:, 3 * cp:3 * cp + c]
        taps = _taps_2d(hp, psc, ml, mr, w, pad, hw)
        pooled = functools.reduce(lambda x, y: x + y, taps) * (1.0 / 9.0)
        hp_ref[j] = pooled.astype(jnp.bfloat16)
        s1 = _acc(s1, jnp.sum(hm, axis=0, keepdims=True))
        ss1 = _acc(ss1, jnp.sum(hm * hm, axis=0, keepdims=True))
        s3 = _acc(s3, jnp.sum(pooled, axis=0, keepdims=True))
        ss3 = _acc(ss3, jnp.sum(pooled * pooled, axis=0, keepdims=True))
    s1_ref[0] = s1
    ss1_ref[0] = ss1
    s3_ref[0] = s3
    ss3_ref[0] = ss3


def _conv3x3(a, w_ref, scratch, ml, mr, w, pad, hw):
    taps = _taps_2d(a, scratch, ml, mr, w, pad, hw)
    im = jnp.concatenate(taps, axis=1)
    return jnp.dot(im, w_ref[...], preferred_element_type=jnp.float32)


def _k2(x_ref, wm_ref, hp_ref, s1_ref, ss1_ref, s3_ref, ss3_ref,
        g1_ref, b1_ref, g3_ref, b3_ref, w01_ref, w11_ref, mlb_ref, mrb_ref,
        c0_ref, c1_ref, o2_ref, o3_ref,
        sc0_ref, ssc0_ref, sc1_ref, ssc1_ref,
        sca, scb, *, w, hw, pad, cp, c, count):
    scale, offset = _finalize_bn(s1_ref, ss1_ref, g1_ref, b1_ref, count)
    sc3, of3 = _finalize_bn(s3_ref, ss3_ref, g3_ref, b3_ref, count)
    ml = mlb_ref[...]
    mr = mrb_ref[...]
    sc0 = ssc0 = sc1 = ssc1 = None
    for j in range(_IPP):
        xb = x_ref[j].astype(jnp.bfloat16)
        hm = lax.dot_general(xb, wm_ref[...], (((0,), (0,)), ((), ())),
                             preferred_element_type=jnp.float32)
        act = jnp.maximum(hm * scale + offset, 0.0)
        actb = act.astype(jnp.bfloat16)
        o2_ref[j] = actb[:, 2 * cp:2 * cp + c]
        o3_ref[j] = jnp.maximum(
            hp_ref[j].astype(jnp.float32) * sc3 + of3,
            0.0).astype(jnp.bfloat16)
        c0 = _conv3x3(actb[:, 0:cp], w01_ref, sca, ml, mr, w, pad, hw)
        c0_ref[j] = c0.astype(jnp.bfloat16)
        sc0 = _acc(sc0, jnp.sum(c0, axis=0, keepdims=True))
        ssc0 = _acc(ssc0, jnp.sum(c0 * c0, axis=0, keepdims=True))
        c1 = _conv3x3(actb[:, cp:2 * cp], w11_ref, scb, ml, mr, w, pad, hw)
        c1_ref[j] = c1.astype(jnp.bfloat16)
        sc1 = _acc(sc1, jnp.sum(c1, axis=0, keepdims=True))
        ssc1 = _acc(ssc1, jnp.sum(c1 * c1, axis=0, keepdims=True))
    sc0_ref[0] = sc0
    ssc0_ref[0] = ssc0
    sc1_ref[0] = sc1
    ssc1_ref[0] = ssc1


def _k3(c0_ref, c1_ref, sc0_ref, ssc0_ref, sc1_ref, ssc1_ref,
        g01_ref, b01_ref, g11_ref, b11_ref, w12_ref, mlb_ref, mrb_ref,
        o0_ref, c2_ref, sc2_ref, ssc2_ref, scr, *, w, hw, pad, cp, count):
    s0, o0 = _finalize_bn(sc0_ref, ssc0_ref, g01_ref, b01_ref, count)
    s1, of1 = _finalize_bn(sc1_ref, ssc1_ref, g11_ref, b11_ref, count)
    ml = mlb_ref[...]
    mr = mrb_ref[...]
    sc2 = ssc2 = None
    for j in range(_IPP):
        o0_ref[j] = jnp.maximum(
            c0_ref[j].astype(jnp.float32) * s0 + o0, 0.0).astype(jnp.bfloat16)
        act = jnp.maximum(
            c1_ref[j].astype(jnp.float32) * s1 + of1,
            0.0).astype(jnp.bfloat16)
        c2 = _conv3x3(act, w12_ref, scr, ml, mr, w, pad, hw)
        c2_ref[j] = c2.astype(jnp.bfloat16)
        sc2 = _acc(sc2, jnp.sum(c2, axis=0, keepdims=True))
        ssc2 = _acc(ssc2, jnp.sum(c2 * c2, axis=0, keepdims=True))
    sc2_ref[0] = sc2
    ssc2_ref[0] = ssc2


def _k4(o0_ref, c2_ref, o2_ref, o3_ref, sc2_ref, ssc2_ref,
        g12_ref, b12_ref, out_ref, *, c, count):
    s2, of2 = _finalize_bn(sc2_ref, ssc2_ref, g12_ref, b12_ref, count)
    for j in range(_IPP):
        a1 = jnp.maximum(c2_ref[j].astype(jnp.float32) * s2 + of2, 0.0)
        out_ref[j, 0:c, :] = jnp.transpose(
            o0_ref[j].astype(jnp.float32), (1, 0))
        out_ref[j, c:2 * c, :] = jnp.transpose(a1, (1, 0))
        out_ref[j, 2 * c:3 * c, :] = jnp.transpose(
            o2_ref[j].astype(jnp.float32), (1, 0))
        out_ref[j, 3 * c:4 * c, :] = jnp.transpose(
            o3_ref[j].astype(jnp.float32), (1, 0))


def _pad_vec(v, cc):
    return jnp.zeros((1, cc), jnp.float32).at[0, :v.shape[0]].set(v)


def kernel(x_nchw, b00_w, b00_g, b00_b, b01_w, b01_g, b01_b,
           b10_w, b10_g, b10_b, b11_w, b11_g, b11_b, b12_w, b12_g, b12_b,
           b2_w, b2_g, b2_b, b3_w, b3_g, b3_b):
    n, cin, h, w = x_nchw.shape
    c = b00_w.shape[-1]
    cp = 128
    hw = h * w
    pad = w + 1
    count = float(n * hw)
    bf = jnp.bfloat16
    f32 = jnp.float32
    ipp = _IPP
    ng = n // ipp

    x3 = x_nchw.reshape(n, cin, hw)

    wcol = (jnp.arange(hw, dtype=jnp.int32) % w).reshape(hw, 1)
    mlf = (wcol >= 1).astype(f32)
    mrf = (wcol <= w - 2).astype(f32)
    mlb = mlf.astype(bf)
    mrb = mrf.astype(bf)

    wall = (jnp.zeros((cin, 4 * cp), f32)
            .at[:, 0:c].set(b00_w.reshape(cin, c))
            .at[:, cp:cp + c].set(b10_w.reshape(cin, c))
            .at[:, 2 * cp:2 * cp + c].set(b2_w.reshape(cin, c))
            .at[:, 3 * cp:3 * cp + c].set(b3_w.reshape(cin, c))).astype(bf)
    wm = wall[:, 0:3 * cp]
    g1 = (jnp.zeros((1, 3 * cp), f32).at[0, 0:c].set(b00_g)
          .at[0, cp:cp + c].set(b10_g).at[0, 2 * cp:2 * cp + c].set(b2_g))
    b1 = (jnp.zeros((1, 3 * cp), f32).at[0, 0:c].set(b00_b)
          .at[0, cp:cp + c].set(b10_b).at[0, 2 * cp:2 * cp + c].set(b2_b))
    w01 = (jnp.zeros((3, 3, cp, c), f32).at[:, :, :c, :].set(b01_w)
           .reshape(9 * cp, c)).astype(bf)
    w11 = (jnp.zeros((3, 3, cp, cp), f32).at[:, :, :c, :c].set(b11_w)
           .reshape(9 * cp, cp)).astype(bf)
    w12 = (jnp.zeros((3, 3, cp, c), f32).at[:, :, :c, :].set(b12_w)
           .reshape(9 * cp, c)).astype(bf)

    par = pltpu.CompilerParams(dimension_semantics=("parallel",))
    img = lambda cc: pl.BlockSpec((ipp, hw, cc), lambda i: (i, 0, 0))
    istat = lambda cc: pl.BlockSpec((1, 1, cc), lambda i: (i, 0, 0))
    stat = lambda cc: pl.BlockSpec((ng, 1, cc), lambda i: (0, 0, 0))
    vec = lambda cc: pl.BlockSpec((1, cc), lambda i: (0, 0))
    mask = pl.BlockSpec((hw, 1), lambda i: (0, 0))
    xspec = pl.BlockSpec((ipp, cin, hw), lambda i: (i, 0, 0))

    hp, s1, ss1, s3, ss3 = pl.pallas_call(
        functools.partial(_k1, w=w, hw=hw, pad=pad, cp=cp, c=c),
        grid=(ng,),
        in_specs=[xspec, pl.BlockSpec((cin, 4 * cp), lambda i: (0, 0)),
                  mask, mask],
        out_specs=(img(c), istat(3 * cp), istat(3 * cp), istat(c), istat(c)),
        out_shape=(jax.ShapeDtypeStruct((n, hw, c), bf),
                   jax.ShapeDtypeStruct((ng, 1, 3 * cp), f32),
                   jax.ShapeDtypeStruct((ng, 1, 3 * cp), f32),
                   jax.ShapeDtypeStruct((ng, 1, c), f32),
                   jax.ShapeDtypeStruct((ng, 1, c), f32)),
        scratch_shapes=[pltpu.VMEM((hw + 2 * pad, c), f32)],
        compiler_params=par,
    )(x3, wall, mlf, mrf)

    c0, c1, o2, o3, sc0, ssc0, sc1, ssc1 = pl.pallas_call(
        functools.partial(_k2, w=w, hw=hw, pad=pad, cp=cp, c=c, count=count),
        grid=(ng,),
        in_specs=[xspec, pl.BlockSpec((cin, 3 * cp), lambda i: (0, 0)),
                  img(c), stat(3 * cp), stat(3 * cp), stat(c), stat(c),
                  vec(3 * cp), vec(3 * cp), vec(c), vec(c),
                  pl.BlockSpec((9 * cp, c), lambda i: (0, 0)),
                  pl.BlockSpec((9 * cp, cp), lambda i: (0, 0)),
                  mask, mask],
        out_specs=(img(c), img(cp), img(c), img(c),
                   istat(c), istat(c), istat(cp), istat(cp)),
        out_shape=(jax.ShapeDtypeStruct((n, hw, c), bf),
                   jax.ShapeDtypeStruct((n, hw, cp), bf),
                   jax.ShapeDtypeStruct((n, hw, c), bf),
                   jax.ShapeDtypeStruct((n, hw, c), bf),
                   jax.ShapeDtypeStruct((ng, 1, c), f32),
                   jax.ShapeDtypeStruct((ng, 1, c), f32),
                   jax.ShapeDtypeStruct((ng, 1, cp), f32),
                   jax.ShapeDtypeStruct((ng, 1, cp), f32)),
        scratch_shapes=[pltpu.VMEM((hw + 2 * pad, cp), bf),
                        pltpu.VMEM((hw + 2 * pad, cp), bf)],
        compiler_params=par,
    )(x3, wm, hp, s1, ss1, s3, ss3, g1, b1,
      b3_g.reshape(1, c), b3_b.reshape(1, c), w01, w11, mlb, mrb)

    g11 = _pad_vec(b11_g, cp)
    b11 = _pad_vec(b11_b, cp)
    o0, c2, sc2, ssc2 = pl.pallas_call(
        functools.partial(_k3, w=w, hw=hw, pad=pad, cp=cp, count=count),
        grid=(ng,),
        in_specs=[img(c), img(cp),
                  stat(c), stat(c), stat(cp), stat(cp),
                  vec(c), vec(c), vec(cp), vec(cp),
                  pl.BlockSpec((9 * cp, c), lambda i: (0, 0)),
                  mask, mask],
        out_specs=(img(c), img(c), istat(c), istat(c)),
        out_shape=(jax.ShapeDtypeStruct((n, hw, c), bf),
                   jax.ShapeDtypeStruct((n, hw, c), bf),
                   jax.ShapeDtypeStruct((ng, 1, c), f32),
                   jax.ShapeDtypeStruct((ng, 1, c), f32)),
        scratch_shapes=[pltpu.VMEM((hw + 2 * pad, cp), bf)],
        compiler_params=par,
    )(c0, c1, sc0, ssc0, sc1, ssc1,
      b01_g.reshape(1, c), b01_b.reshape(1, c), g11, b11, w12, mlb, mrb)

    out = pl.pallas_call(
        functools.partial(_k4, c=c, count=count),
        grid=(ng,),
        in_specs=[img(c), img(c), img(c), img(c),
                  stat(c), stat(c), vec(c), vec(c)],
        out_specs=pl.BlockSpec((ipp, 4 * c, hw), lambda i: (i, 0, 0)),
        out_shape=jax.ShapeDtypeStruct((n, 4 * c, hw), f32),
        compiler_params=par,
    )(o0, c2, o2, o3, sc2, ssc2, b12_g.reshape(1, c), b12_b.reshape(1, c))

    return out.reshape(n, 4 * c, h, w)
